# baseline (speedup 1.0000x reference)
import jax
import jax.numpy as jnp
from jax.experimental import pallas as pl
from jax.experimental.pallas import tpu as pltpu

_B, _N, _D = 32, 100, 128
_BN = _B * _N
_INV_DEG = 1.0 / (_N + 1)


def _encoder_kernel(x_ref, wi_ref, bi_ref, w0_ref, b0_ref, w1_ref, b1_ref,
                    w2_ref, b2_ref, upd_ref, nf_ref,
                    nf_vmem, upd_vmem, nf_sem, upd_sem):
    nf = jnp.dot(x_ref[...], wi_ref[...], preferred_element_type=jnp.float32)
    nf = nf + bi_ref[...]
    nf_vmem[...] = nf
    nf_copy = pltpu.make_async_copy(nf_vmem, nf_ref, nf_sem)
    nf_copy.start()

    lane_i = jax.lax.broadcasted_iota(jnp.int32, (_B, _BN), 1)
    lo = jax.lax.broadcasted_iota(jnp.int32, (_B, _BN), 0) * _N
    sel = jnp.where((lane_i >= lo) & (lane_i < lo + _N), 1.0, 0.0)
    row_i = jax.lax.broadcasted_iota(jnp.int32, (_BN, _B), 0)
    lo_t = jax.lax.broadcasted_iota(jnp.int32, (_BN, _B), 1) * _N
    sel_t = jnp.where((row_i >= lo_t) & (row_i < lo_t + _N), 1.0, 0.0)

    h = nf
    for w_ref, b_ref, relu in ((w0_ref, b0_ref, True),
                               (w1_ref, b1_ref, True),
                               (w2_ref, b2_ref, False)):
        xw = jnp.dot(h, w_ref[...] * _INV_DEG,
                     preferred_element_type=jnp.float32)
        sg = jnp.dot(sel, xw, preferred_element_type=jnp.float32) + b_ref[...]
        h = xw + jnp.dot(sel_t, sg, preferred_element_type=jnp.float32)
        if relu:
            h = jnp.maximum(h, 0.0)
    m = jnp.max(h, axis=1, keepdims=True)
    e = h - m
    lse = jnp.log(jnp.sum(jnp.exp(e), axis=1, keepdims=True))
    upd_vmem[...] = e + (nf - lse)
    upd_copy = pltpu.make_async_copy(upd_vmem, upd_ref, upd_sem)
    upd_copy.start()
    nf_copy.wait()
    upd_copy.wait()


def kernel(x, W_init, b_init, W0, b0, W1, b1, W2, b2):
    x2 = x.reshape(_BN, 2)
    b_init = b_init.reshape(1, _D)
    b0 = b0.reshape(1, _D)
    b1 = b1.reshape(1, _D)
    b2 = b2.reshape(1, _D)

    out_shape = jax.ShapeDtypeStruct((_BN, _D), jnp.float32)
    hbm = pl.BlockSpec(memory_space=pltpu.MemorySpace.HBM)
    update, node_feature = pl.pallas_call(
        _encoder_kernel,
        out_shape=[out_shape, out_shape],
        out_specs=[hbm, hbm],
        scratch_shapes=[
            pltpu.VMEM((_BN, _D), jnp.float32),
            pltpu.VMEM((_BN, _D), jnp.float32),
            pltpu.SemaphoreType.DMA,
            pltpu.SemaphoreType.DMA,
        ],
    )(x2, W_init, b_init, W0, b0, W1, b1, W2, b2)
    return (update.reshape(_B, _N, _D), node_feature.reshape(_B, _N, _D))
